# baseline (speedup 1.0000x reference)
import functools

import jax
import jax.numpy as jnp
from jax import lax
from jax.experimental import pallas as pl
from jax.experimental.pallas import tpu as pltpu
from jax.experimental.pallas import tpu_sc as plsc

_NC = 2
_NS = 16
_NW = _NC * _NS
_LANES = 16


def _matmul(x, wr, b2, s):
    n, c = x.shape
    co = wr.shape[1] // s

    def body(x_ref, w_ref, b_ref, o_ref):
        z = jnp.dot(x_ref[...], w_ref[...], preferred_element_type=jnp.float32)

        @pl.when(pl.program_id(0) == 0)
        def _():
            o_ref[...] = (z + b_ref[...])[None]

        @pl.when(pl.program_id(0) != 0)
        def _():
            o_ref[...] = z[None]

    return pl.pallas_call(
        body,
        grid=(s,),
        in_specs=[
            pl.BlockSpec((n, c), lambda i: (0, 0)),
            pl.BlockSpec((c, co), lambda i: (0, i)),
            pl.BlockSpec((1, co), lambda i: (0, 0)),
        ],
        out_specs=pl.BlockSpec((1, n, co), lambda i: (i, 0, 0)),
        out_shape=jax.ShapeDtypeStruct((s, n, co), jnp.float32),
    )(x, wr, b2)


def _sc_gather_sum(zf, idxt3, np_, n, s, co, ch):
    blk = s * ch
    mesh = plsc.VectorSubcoreMesh(
        core_axis_name="cx", subcore_axis_name="sx", num_cores=_NC,
        num_subcores=_NS)

    @functools.partial(
        pl.kernel,
        out_type=jax.ShapeDtypeStruct((np_, co), jnp.float32),
        mesh=mesh,
        scratch_types=[
            pltpu.VMEM((blk,), jnp.int32),
            pltpu.VMEM((ch, co), jnp.float32),
            pltpu.SemaphoreType.DMA,
        ],
    )
    def run(z_hbm, idxt_hbm, out_hbm, idx_v, acc_v, sem):
        cid = lax.axis_index("cx")
        sid = lax.axis_index("sx")

        @pl.when(cid == 0)
        def _():
            pltpu.sync_copy(
                idxt_hbm.at[pl.ds(pl.multiple_of(sid * blk, blk), blk)], idx_v)

            def gouter(sv, _):
                def gbody(i, _):
                    p = sv * ch + i * _LANES
                    idx_v[pl.ds(p, _LANES)] = idx_v[pl.ds(p, _LANES)] + sv * n
                    return 0
                lax.fori_loop(0, ch // _LANES, gbody, 0, unroll=4)
                return 0

            lax.fori_loop(1, s, gouter, 0)

            pltpu.async_copy(z_hbm.at[idx_v.at[pl.ds(0, ch)]], acc_v, sem).wait()

            def fire(sv, _):
                pltpu.async_copy(
                    z_hbm.at[idx_v.at[pl.ds(pl.multiple_of(sv * ch, ch), ch)]],
                    acc_v, sem, add=True)
                return 0

            lax.fori_loop(1, s, fire, 0)

            def drain(sv, _):
                pltpu.make_async_copy(z_hbm.at[pl.ds(0, ch)], acc_v, sem).wait()
                return 0

            lax.fori_loop(1, s, drain, 0)

            pltpu.sync_copy(
                acc_v, out_hbm.at[pl.ds(pl.multiple_of(sid * ch, ch), ch)])

    return run(zf, idxt3)


def kernel(inputs, indices, W, b):
    batch, n, c = inputs.shape
    n_nodes, s = indices.shape
    co = W.shape[1]

    x = inputs.reshape(n, c)
    wr = W.reshape(s, c, co).transpose(1, 0, 2).reshape(c, s * co)

    z3 = _matmul(x, wr, b.reshape(1, co), s)
    zf = z3.reshape(s * n, co)

    ch = -(-n // _NS)
    ch = -(-ch // _LANES) * _LANES
    np_ = _NS * ch
    idx = indices.astype(jnp.int32)
    idxp = jnp.pad(idx, ((0, np_ - n), (0, 0)))
    idxt3 = idxp.reshape(_NS, ch, s).transpose(0, 2, 1).reshape(-1)

    outp = _sc_gather_sum(zf, idxt3, np_, n, s, co, ch)
    return outp[:n].reshape(batch, n, co)


# speedup vs baseline: 1.0012x; 1.0012x over previous
import functools

import jax
import jax.numpy as jnp
from jax import lax
from jax.experimental import pallas as pl
from jax.experimental.pallas import tpu as pltpu
from jax.experimental.pallas import tpu_sc as plsc

_NC = 2
_NS = 16
_NW = _NC * _NS
_LANES = 16


def _matmul(x, wr, b2, s):
    n, c = x.shape
    co = wr.shape[1] // s

    def body(x_ref, w_ref, b_ref, o_ref):
        z = jnp.dot(x_ref[...], w_ref[...], preferred_element_type=jnp.float32)

        @pl.when(pl.program_id(0) == 0)
        def _():
            o_ref[...] = (z + b_ref[...])[None]

        @pl.when(pl.program_id(0) != 0)
        def _():
            o_ref[...] = z[None]

    return pl.pallas_call(
        body,
        grid=(s,),
        in_specs=[
            pl.BlockSpec((n, c), lambda i: (0, 0)),
            pl.BlockSpec((c, co), lambda i: (0, i)),
            pl.BlockSpec((1, co), lambda i: (0, 0)),
        ],
        out_specs=pl.BlockSpec((1, n, co), lambda i: (i, 0, 0)),
        out_shape=jax.ShapeDtypeStruct((s, n, co), jnp.float32),
    )(x, wr, b2)


def _sc_gather_sum(zf, idxt3, np_, n, s, co, ch):
    blk = s * ch
    mesh = plsc.VectorSubcoreMesh(
        core_axis_name="cx", subcore_axis_name="sx", num_cores=_NC,
        num_subcores=_NS)

    @functools.partial(
        pl.kernel,
        out_type=jax.ShapeDtypeStruct((np_, co), jnp.float32),
        mesh=mesh,
        scratch_types=[
            pltpu.VMEM((blk,), jnp.int32),
            pltpu.VMEM((ch, co), jnp.float32),
            pltpu.SemaphoreType.DMA,
        ],
    )
    def run(z_hbm, idxt_hbm, out_hbm, idx_v, acc_v, sem):
        cid = lax.axis_index("cx")
        sid = lax.axis_index("sx")

        hf = ch // 2

        @pl.when(cid == 0)
        def _():
            pltpu.sync_copy(
                idxt_hbm.at[pl.ds(pl.multiple_of(sid * blk, blk), blk)], idx_v)

            def gouter(sv, _):
                def gbody(i, _):
                    p = sv * ch + i * _LANES
                    idx_v[pl.ds(p, _LANES)] = idx_v[pl.ds(p, _LANES)] + sv * n
                    return 0
                lax.fori_loop(0, ch // _LANES, gbody, 0, unroll=4)
                return 0

            lax.fori_loop(1, s, gouter, 0)

            d0 = pltpu.async_copy(
                z_hbm.at[idx_v.at[pl.ds(0, hf)]], acc_v.at[pl.ds(0, hf)], sem)
            d1 = pltpu.async_copy(
                z_hbm.at[idx_v.at[pl.ds(hf, hf)]], acc_v.at[pl.ds(hf, hf)], sem)
            d0.wait()
            d1.wait()

            def fire(sv, _):
                off = pl.multiple_of(sv * ch, ch)
                pltpu.async_copy(z_hbm.at[idx_v.at[pl.ds(off, hf)]],
                                 acc_v.at[pl.ds(0, hf)], sem, add=True)
                pltpu.async_copy(z_hbm.at[idx_v.at[pl.ds(off + hf, hf)]],
                                 acc_v.at[pl.ds(hf, hf)], sem, add=True)
                return 0

            lax.fori_loop(1, s, fire, 0)

            def drain(sv, _):
                pltpu.make_async_copy(
                    z_hbm.at[pl.ds(0, hf)], acc_v.at[pl.ds(0, hf)], sem).wait()
                return 0

            lax.fori_loop(0, 2 * (s - 1), drain, 0)

            pltpu.sync_copy(
                acc_v, out_hbm.at[pl.ds(pl.multiple_of(sid * ch, ch), ch)])

    return run(zf, idxt3)


def kernel(inputs, indices, W, b):
    batch, n, c = inputs.shape
    n_nodes, s = indices.shape
    co = W.shape[1]

    x = inputs.reshape(n, c)
    wr = W.reshape(s, c, co).transpose(1, 0, 2).reshape(c, s * co)

    z3 = _matmul(x, wr, b.reshape(1, co), s)
    zf = z3.reshape(s * n, co)

    ch = -(-n // _NS)
    ch = -(-ch // _LANES) * _LANES
    np_ = _NS * ch
    idx = indices.astype(jnp.int32)
    idxp = jnp.pad(idx, ((0, np_ - n), (0, 0)))
    idxt3 = idxp.reshape(_NS, ch, s).transpose(0, 2, 1).reshape(-1)

    outp = _sc_gather_sum(zf, idxt3, np_, n, s, co, ch)
    return outp[:n].reshape(batch, n, co)
